# baseline (speedup 1.0000x reference)
import functools

import jax
import jax.numpy as jnp
from jax import lax
from jax.experimental import pallas as pl
from jax.experimental.pallas import tpu as pltpu
from jax.experimental.pallas import tpu_sc as plsc

HID = 2048
BS = 2048
NE = 16
TOPK = 2
INTER = 448
PAIRS = BS * TOPK
BLK = 256
NB = PAIRS // BLK + NE
NSLOT = NB * BLK
NWORK = 32
SLOT_PER_W = NSLOT // NWORK
GCH = 16
TOK_PER_W = BS // NWORK
NSPLIT = 2
HNB = NB // NSPLIT
HSLOT = NSLOT // NSPLIT
HALF_PER_W = HSLOT // NWORK

@functools.cache
def _sc_mesh():
    return plsc.VectorSubcoreMesh(
        core_axis_name="c", subcore_axis_name="s", num_cores=2,
        num_subcores=16)


_F32 = jnp.float32
_I32 = jnp.int32


def _router_body(x_ref, gw_ref, slots_ref, wts_ref, be_ref, act_ref):
    x = x_ref[...]
    gw = gw_ref[...]
    logits = lax.dot_general(x, gw, (((1,), (1,)), ((), ())),
                             preferred_element_type=_F32)
    m = jnp.max(logits, axis=1, keepdims=True)
    ex = jnp.exp(logits - m)
    rw = ex / jnp.sum(ex, axis=1, keepdims=True)

    ids = lax.broadcasted_iota(_I32, (BS, NE), 1)
    m1 = jnp.max(rw, axis=1, keepdims=True)
    e1 = jnp.min(jnp.where(rw == m1, ids, NE), axis=1, keepdims=True)
    oh1 = (ids == e1)
    v1 = jnp.sum(jnp.where(oh1, rw, 0.0), axis=1, keepdims=True)
    rw2 = jnp.where(oh1, -1.0, rw)
    m2 = jnp.max(rw2, axis=1, keepdims=True)
    e2 = jnp.min(jnp.where(rw2 == m2, ids, NE), axis=1, keepdims=True)
    oh2 = (ids == e2)
    v2 = jnp.sum(jnp.where(oh2, rw, 0.0), axis=1, keepdims=True)
    s = v1 + v2
    w1 = v1 / s
    w2 = v2 / s

    ind = oh1.astype(_F32) + oh2.astype(_F32)
    ri = lax.broadcasted_iota(_I32, (BS, BS), 0)
    ci = lax.broadcasted_iota(_I32, (BS, BS), 1)
    ltri = (ri > ci).astype(_F32)
    rank = lax.dot_general(ltri, ind, (((1,), (0,)), ((), ())),
                           preferred_element_type=_F32)
    counts = jnp.sum(ind, axis=0, keepdims=True)
    pc = jnp.floor((counts + (BLK - 1)) / BLK) * BLK
    e16r = lax.broadcasted_iota(_I32, (NE, NE), 0)
    e16c = lax.broadcasted_iota(_I32, (NE, NE), 1)
    eye16 = (e16r == e16c).astype(_F32)
    sut16 = (e16r < e16c).astype(_F32)
    starts = lax.dot_general(pc, sut16, (((1,), (0,)), ((), ())),
                             preferred_element_type=_F32)
    total = jnp.sum(pc, axis=1, keepdims=True)

    rank1 = jnp.sum(jnp.where(oh1, rank, 0.0), axis=1, keepdims=True)
    rank2 = jnp.sum(jnp.where(oh2, rank, 0.0), axis=1, keepdims=True)
    st1 = jnp.sum(jnp.where(oh1, starts, 0.0), axis=1, keepdims=True)
    st2 = jnp.sum(jnp.where(oh2, starts, 0.0), axis=1, keepdims=True)
    slots_ref[:, 0:1] = (st1 + rank1).astype(_I32)
    slots_ref[:, 1:2] = (st2 + rank2).astype(_I32)
    wts_ref[:, 0:1] = w1
    wts_ref[:, 1:2] = w2

    starts_c = lax.dot_general(eye16, starts, (((1,), (1,)), ((), ())),
                               preferred_element_type=_F32)
    pc_c = lax.dot_general(eye16, pc, (((1,), (1,)), ((), ())),
                           preferred_element_type=_F32)
    bpos = (lax.broadcasted_iota(_I32, (NE, NB), 1) * BLK).astype(_F32)
    inb = jnp.logical_and(bpos >= starts_c, bpos < starts_c + pc_c)
    eids = lax.broadcasted_iota(_I32, (NE, NB), 0).astype(_F32)
    bef = jnp.sum(jnp.where(inb, eids, 0.0), axis=0, keepdims=True)
    bpos_r = (lax.broadcasted_iota(_I32, (1, NB), 1) * BLK).astype(_F32)
    actf = (bpos_r < total).astype(_F32)
    be_ref[...] = (bef * actf + (1.0 - actf) * (NE - 1)).astype(_I32)
    act_ref[...] = actf.astype(_I32)


def _router_dispatch(x, gate_w):
    return pl.pallas_call(
        _router_body,
        out_shape=[
            jax.ShapeDtypeStruct((BS, TOPK), _I32),
            jax.ShapeDtypeStruct((BS, TOPK), _F32),
            jax.ShapeDtypeStruct((1, NB), _I32),
            jax.ShapeDtypeStruct((1, NB), _I32),
        ],
    )(x, gate_w)


@functools.cache
def _build_sc_scatter():
  @functools.partial(
      pl.kernel,
      out_type=[
          jax.ShapeDtypeStruct((NSLOT,), _I32),
          jax.ShapeDtypeStruct((NSLOT,), _F32),
      ],
      mesh=_sc_mesh(),
      scratch_types=[
          pltpu.VMEM((PAIRS,), _I32),
          pltpu.VMEM((PAIRS,), _F32),
          pltpu.VMEM((NSLOT,), _I32),
          pltpu.VMEM((NSLOT,), _F32),
      ],
      compiler_params=pltpu.CompilerParams(needs_layout_passes=False),
  )
  def _sc_scatter(slots_hbm, w_hbm, tok_hbm, ws_hbm, slots_v, w_v, tok_v,
                  ws_v):
    wid = lax.axis_index("s") * 2 + lax.axis_index("c")

    @pl.when(wid == 0)
    def _():
        pltpu.sync_copy(slots_hbm, slots_v)
        pltpu.sync_copy(w_hbm, w_v)
        zi = jnp.zeros((16,), _I32)
        zf = jnp.zeros((16,), _F32)

        def zbody(i, carry):
            tok_v[pl.ds(i * 16, 16)] = zi
            ws_v[pl.ds(i * 16, 16)] = zf
            return carry

        lax.fori_loop(0, NSLOT // 16, zbody, 0)

        def sbody(i, carry):
            sl = slots_v[pl.ds(i * 16, 16)]
            wv = w_v[pl.ds(i * 16, 16)]
            j = lax.iota(_I32, 16) + i * 16
            tok = lax.shift_right_logical(j, 1)
            plsc.store_scatter(tok_v, [sl], tok)
            plsc.store_scatter(ws_v, [sl], wv)
            return carry

        lax.fori_loop(0, PAIRS // 16, sbody, 0)
        pltpu.sync_copy(tok_v, tok_hbm)
        pltpu.sync_copy(ws_v, ws_hbm)

  return _sc_scatter


@functools.cache
def _build_sc_gather(off):
  @functools.partial(
      pl.kernel,
      out_type=jax.ShapeDtypeStruct((HSLOT, HID), _F32),
      mesh=_sc_mesh(),
      scratch_types=[
          pltpu.VMEM((HALF_PER_W,), _I32),
          pltpu.VMEM((GCH, HID), _F32),
          pltpu.VMEM((GCH, HID), _F32),
          pltpu.SemaphoreType.DMA,
          pltpu.SemaphoreType.DMA,
      ],
      compiler_params=pltpu.CompilerParams(needs_layout_passes=False),
  )
  def _sc_gather(tok_hbm, x_hbm, xs_hbm, tok_v, buf0, buf1, sem0, sem1):
    wid = lax.axis_index("s") * 2 + lax.axis_index("c")
    base = wid * HALF_PER_W
    pltpu.sync_copy(tok_hbm.at[pl.ds(off * HSLOT + base, HALF_PER_W)], tok_v)
    bufs = (buf0, buf1)
    sems = (sem0, sem1)
    nch = HALF_PER_W // GCH
    h = pltpu.async_copy(x_hbm.at[tok_v.at[pl.ds(0, GCH)]], bufs[0], sems[0])
    for c in range(nch):
        h.wait()
        if c + 1 < nch:
            h = pltpu.async_copy(
                x_hbm.at[tok_v.at[pl.ds((c + 1) * GCH, GCH)]],
                bufs[(c + 1) % 2], sems[(c + 1) % 2])
        pltpu.sync_copy(bufs[c % 2], xs_hbm.at[pl.ds(base + c * GCH, GCH)])

  return _sc_gather


def _mlp_body(be_ref, act_ref, xs_ref, u_ref, g_ref, d_ref, w_ref, ys_ref):
    b = pl.program_id(0)

    @pl.when(act_ref[b] == 1)
    def _():
        xb = xs_ref[...]
        hg = lax.dot_general(xb, g_ref[...], (((1,), (1,)), ((), ())),
                             preferred_element_type=_F32,
                             precision=lax.Precision.DEFAULT)
        hu = lax.dot_general(xb, u_ref[...], (((1,), (1,)), ((), ())),
                             preferred_element_type=_F32,
                             precision=lax.Precision.DEFAULT)
        hact = (hg * (1.0 / (1.0 + jnp.exp(-hg)))) * hu
        y = lax.dot_general(hact, d_ref[0], (((1,), (0,)), ((), ())),
                            preferred_element_type=_F32,
                            precision=lax.Precision.DEFAULT)
        ys_ref[...] = y * w_ref[...]


def _mlp_body2(be_ref, act_ref, xs_ref, u_ref, g_ref, d_ref, w_ref, ys_in,
               ys_ref):
    _mlp_body(be_ref, act_ref, xs_ref, u_ref, g_ref, d_ref, w_ref, ys_ref)


def _grouped_mlp_part(off, be, act, xs, u, g, d3, ws, ys_in=None):
    gb = off * HNB
    in_specs = [
        pl.BlockSpec((BLK, HID), lambda b, be, act: (b, 0)),
        pl.BlockSpec((INTER, HID), lambda b, be, act: (be[b + gb], 0)),
        pl.BlockSpec((INTER, HID), lambda b, be, act: (be[b + gb], 0)),
        pl.BlockSpec((1, INTER, HID), lambda b, be, act: (be[b + gb], 0, 0)),
        pl.BlockSpec((BLK, 1), lambda b, be, act: (b + gb, 0)),
    ]
    args = [be, act, xs, u, g, d3, ws]
    body = _mlp_body
    aliases = {}
    if ys_in is not None:
        in_specs.append(pl.BlockSpec(memory_space=pl.ANY))
        args.append(ys_in)
        body = _mlp_body2
        aliases = {7: 0}
    grid_spec = pltpu.PrefetchScalarGridSpec(
        num_scalar_prefetch=2,
        grid=(HNB,),
        in_specs=in_specs,
        out_specs=pl.BlockSpec((BLK, HID), lambda b, be, act: (b + gb, 0)),
    )
    return pl.pallas_call(
        body,
        grid_spec=grid_spec,
        out_shape=jax.ShapeDtypeStruct((NSLOT, HID), _F32),
        input_output_aliases=aliases,
        compiler_params=pltpu.CompilerParams(
            dimension_semantics=("arbitrary",)),
    )(*args)


@functools.cache
def _build_sc_combine():
  @functools.partial(
      pl.kernel,
      out_type=jax.ShapeDtypeStruct((BS, HID), _F32),
      mesh=_sc_mesh(),
      scratch_types=[
          pltpu.VMEM((TOPK * TOK_PER_W,), _I32),
          pltpu.VMEM((16, HID), _F32),
          pltpu.VMEM((16, HID), _F32),
          pltpu.VMEM((8, HID), _F32),
          pltpu.SemaphoreType.DMA,
          pltpu.SemaphoreType.DMA,
      ],
      compiler_params=pltpu.CompilerParams(needs_layout_passes=False),
  )
  def _sc_combine(slots_hbm, ys_hbm, out_hbm, sl_v, bufA, bufB, ob, semA,
                  semB):
    wid = lax.axis_index("s") * 2 + lax.axis_index("c")
    t0 = wid * TOK_PER_W
    pltpu.sync_copy(slots_hbm.at[pl.ds(TOPK * t0, TOPK * TOK_PER_W)], sl_v)
    bufs = (bufA, bufB)
    sems = (semA, semB)
    nch = TOK_PER_W // 8
    h = pltpu.async_copy(ys_hbm.at[sl_v.at[pl.ds(0, 16)]], bufs[0], sems[0])
    for c in range(nch):
        h.wait()
        if c + 1 < nch:
            h = pltpu.async_copy(
                ys_hbm.at[sl_v.at[pl.ds((c + 1) * 16, 16)]],
                bufs[(c + 1) % 2], sems[(c + 1) % 2])
        buf = bufs[c % 2]
        for r in range(8):
            def vbody(v, carry, _r=r, _buf=buf):
                a = _buf[2 * _r, pl.ds(v * 16, 16)]
                bq = _buf[2 * _r + 1, pl.ds(v * 16, 16)]
                ob[_r, pl.ds(v * 16, 16)] = a + bq
                return carry

            lax.fori_loop(0, HID // 16, vbody, 0)
        pltpu.sync_copy(ob, out_hbm.at[pl.ds(t0 + c * 8, 8)])

  return _sc_combine


def kernel(hid, gate_w, u, g, d):
    x = hid.reshape(BS, HID)
    slots2, wts2, be2, act2 = _router_dispatch(x, gate_w)
    slots_flat = slots2.reshape(PAIRS)
    w_flat = wts2.reshape(PAIRS)
    be = be2.reshape(NB)
    act = act2.reshape(NB)
    tok, ws = _build_sc_scatter()(slots_flat, w_flat)
    d3 = jnp.transpose(d.reshape(HID, NE, INTER), (1, 2, 0))
    ws2 = ws.reshape(NSLOT, 1)
    xs0 = _build_sc_gather(0)(tok, x)
    xs1 = _build_sc_gather(1)(tok, x)
    ys = _grouped_mlp_part(0, be, act, xs0, u, g, d3, ws2)
    ys = _grouped_mlp_part(1, be, act, xs1, u, g, d3, ws2, ys_in=ys)
    out = _build_sc_combine()(slots_flat, ys)
    return out.reshape(1, BS, HID)


# speedup vs baseline: 1.2387x; 1.2387x over previous
import functools

import jax
import jax.numpy as jnp
from jax import lax
from jax.experimental import pallas as pl
from jax.experimental.pallas import tpu as pltpu
from jax.experimental.pallas import tpu_sc as plsc

HID = 2048
BS = 2048
NE = 16
TOPK = 2
INTER = 448
PAIRS = BS * TOPK
BLK = 128
NB = PAIRS // BLK + NE
NSLOT = NB * BLK
NWORK = 32
SLOT_PER_W = NSLOT // NWORK
GCH = 16
TOK_PER_W = BS // NWORK
NSPLIT = 2
HNB = NB // NSPLIT
HSLOT = NSLOT // NSPLIT
HALF_PER_W = HSLOT // NWORK

@functools.cache
def _sc_mesh():
    return plsc.VectorSubcoreMesh(
        core_axis_name="c", subcore_axis_name="s", num_cores=2,
        num_subcores=16)


_F32 = jnp.float32
_I32 = jnp.int32


def _router_body(x_ref, gw_ref, slots_ref, wts_ref, be_ref, act_ref):
    x = x_ref[...]
    gw = gw_ref[...]
    logits = lax.dot_general(x, gw, (((1,), (1,)), ((), ())),
                             preferred_element_type=_F32)
    m = jnp.max(logits, axis=1, keepdims=True)
    ex = jnp.exp(logits - m)
    rw = ex / jnp.sum(ex, axis=1, keepdims=True)

    ids = lax.broadcasted_iota(_I32, (BS, NE), 1)
    m1 = jnp.max(rw, axis=1, keepdims=True)
    e1 = jnp.min(jnp.where(rw == m1, ids, NE), axis=1, keepdims=True)
    oh1 = (ids == e1)
    v1 = jnp.sum(jnp.where(oh1, rw, 0.0), axis=1, keepdims=True)
    rw2 = jnp.where(oh1, -1.0, rw)
    m2 = jnp.max(rw2, axis=1, keepdims=True)
    e2 = jnp.min(jnp.where(rw2 == m2, ids, NE), axis=1, keepdims=True)
    oh2 = (ids == e2)
    v2 = jnp.sum(jnp.where(oh2, rw, 0.0), axis=1, keepdims=True)
    s = v1 + v2
    w1 = v1 / s
    w2 = v2 / s

    ind = oh1.astype(_F32) + oh2.astype(_F32)
    ri = lax.broadcasted_iota(_I32, (BS, BS), 0)
    ci = lax.broadcasted_iota(_I32, (BS, BS), 1)
    ltri = (ri > ci).astype(_F32)
    rank = lax.dot_general(ltri, ind, (((1,), (0,)), ((), ())),
                           preferred_element_type=_F32)
    counts = jnp.sum(ind, axis=0, keepdims=True)
    pc = jnp.floor((counts + (BLK - 1)) / BLK) * BLK
    e16r = lax.broadcasted_iota(_I32, (NE, NE), 0)
    e16c = lax.broadcasted_iota(_I32, (NE, NE), 1)
    eye16 = (e16r == e16c).astype(_F32)
    sut16 = (e16r < e16c).astype(_F32)
    starts = lax.dot_general(pc, sut16, (((1,), (0,)), ((), ())),
                             preferred_element_type=_F32)
    total = jnp.sum(pc, axis=1, keepdims=True)

    rank1 = jnp.sum(jnp.where(oh1, rank, 0.0), axis=1, keepdims=True)
    rank2 = jnp.sum(jnp.where(oh2, rank, 0.0), axis=1, keepdims=True)
    st1 = jnp.sum(jnp.where(oh1, starts, 0.0), axis=1, keepdims=True)
    st2 = jnp.sum(jnp.where(oh2, starts, 0.0), axis=1, keepdims=True)
    slots_ref[:, 0:1] = (st1 + rank1).astype(_I32)
    slots_ref[:, 1:2] = (st2 + rank2).astype(_I32)
    wts_ref[:, 0:1] = w1
    wts_ref[:, 1:2] = w2

    starts_c = lax.dot_general(eye16, starts, (((1,), (1,)), ((), ())),
                               preferred_element_type=_F32)
    pc_c = lax.dot_general(eye16, pc, (((1,), (1,)), ((), ())),
                           preferred_element_type=_F32)
    bpos = (lax.broadcasted_iota(_I32, (NE, NB), 1) * BLK).astype(_F32)
    inb = jnp.logical_and(bpos >= starts_c, bpos < starts_c + pc_c)
    eids = lax.broadcasted_iota(_I32, (NE, NB), 0).astype(_F32)
    bef = jnp.sum(jnp.where(inb, eids, 0.0), axis=0, keepdims=True)
    bpos_r = (lax.broadcasted_iota(_I32, (1, NB), 1) * BLK).astype(_F32)
    actf = (bpos_r < total).astype(_F32)
    be_ref[...] = (bef * actf + (1.0 - actf) * (NE - 1)).astype(_I32)
    act_ref[...] = actf.astype(_I32)


def _router_dispatch(x, gate_w):
    return pl.pallas_call(
        _router_body,
        out_shape=[
            jax.ShapeDtypeStruct((BS, TOPK), _I32),
            jax.ShapeDtypeStruct((BS, TOPK), _F32),
            jax.ShapeDtypeStruct((1, NB), _I32),
            jax.ShapeDtypeStruct((1, NB), _I32),
        ],
    )(x, gate_w)


@functools.cache
def _build_sc_scatter():
  @functools.partial(
      pl.kernel,
      out_type=[
          jax.ShapeDtypeStruct((NSLOT,), _I32),
          jax.ShapeDtypeStruct((NSLOT,), _F32),
      ],
      mesh=_sc_mesh(),
      scratch_types=[
          pltpu.VMEM((PAIRS,), _I32),
          pltpu.VMEM((PAIRS,), _F32),
          pltpu.VMEM((NSLOT,), _I32),
          pltpu.VMEM((NSLOT,), _F32),
      ],
      compiler_params=pltpu.CompilerParams(needs_layout_passes=False),
  )
  def _sc_scatter(slots_hbm, w_hbm, tok_hbm, ws_hbm, slots_v, w_v, tok_v,
                  ws_v):
    wid = lax.axis_index("s") * 2 + lax.axis_index("c")

    @pl.when(wid == 0)
    def _():
        pltpu.sync_copy(slots_hbm, slots_v)
        pltpu.sync_copy(w_hbm, w_v)
        zi = jnp.zeros((16,), _I32)
        zf = jnp.zeros((16,), _F32)

        def zbody(i, carry):
            tok_v[pl.ds(i * 16, 16)] = zi
            ws_v[pl.ds(i * 16, 16)] = zf
            return carry

        lax.fori_loop(0, NSLOT // 16, zbody, 0)

        def sbody(i, carry):
            sl = slots_v[pl.ds(i * 16, 16)]
            wv = w_v[pl.ds(i * 16, 16)]
            j = lax.iota(_I32, 16) + i * 16
            tok = lax.shift_right_logical(j, 1)
            plsc.store_scatter(tok_v, [sl], tok)
            plsc.store_scatter(ws_v, [sl], wv)
            return carry

        lax.fori_loop(0, PAIRS // 16, sbody, 0)
        pltpu.sync_copy(tok_v, tok_hbm)
        pltpu.sync_copy(ws_v, ws_hbm)

  return _sc_scatter


@functools.cache
def _build_sc_gather(off):
  @functools.partial(
      pl.kernel,
      out_type=jax.ShapeDtypeStruct((HSLOT, HID), _F32),
      mesh=_sc_mesh(),
      scratch_types=[
          pltpu.VMEM((HALF_PER_W,), _I32),
          pltpu.VMEM((GCH, HID), _F32),
          pltpu.VMEM((GCH, HID), _F32),
          pltpu.SemaphoreType.DMA,
          pltpu.SemaphoreType.DMA,
      ],
      compiler_params=pltpu.CompilerParams(needs_layout_passes=False),
  )
  def _sc_gather(tok_hbm, x_hbm, xs_hbm, tok_v, buf0, buf1, sem0, sem1):
    wid = lax.axis_index("s") * 2 + lax.axis_index("c")
    base = wid * HALF_PER_W
    pltpu.sync_copy(tok_hbm.at[pl.ds(off * HSLOT + base, HALF_PER_W)], tok_v)
    bufs = (buf0, buf1)
    sems = (sem0, sem1)
    nch = HALF_PER_W // GCH
    h = pltpu.async_copy(x_hbm.at[tok_v.at[pl.ds(0, GCH)]], bufs[0], sems[0])
    for c in range(nch):
        h.wait()
        if c + 1 < nch:
            h = pltpu.async_copy(
                x_hbm.at[tok_v.at[pl.ds((c + 1) * GCH, GCH)]],
                bufs[(c + 1) % 2], sems[(c + 1) % 2])
        pltpu.sync_copy(bufs[c % 2], xs_hbm.at[pl.ds(base + c * GCH, GCH)])

  return _sc_gather


def _mlp_body(be_ref, act_ref, xs_ref, u_ref, g_ref, d_ref, w_ref, ys_ref):
    b = pl.program_id(0)

    @pl.when(act_ref[b] == 1)
    def _():
        xb = xs_ref[...]
        hg = lax.dot_general(xb, g_ref[...], (((1,), (1,)), ((), ())),
                             preferred_element_type=_F32,
                             precision=lax.Precision.DEFAULT)
        hu = lax.dot_general(xb, u_ref[...], (((1,), (1,)), ((), ())),
                             preferred_element_type=_F32,
                             precision=lax.Precision.DEFAULT)
        hact = (hg * (1.0 / (1.0 + jnp.exp(-hg)))) * hu
        y = lax.dot_general(hact, d_ref[0], (((1,), (0,)), ((), ())),
                            preferred_element_type=_F32,
                            precision=lax.Precision.DEFAULT)
        ys_ref[...] = y * w_ref[...]


def _mlp_body2(be_ref, act_ref, xs_ref, u_ref, g_ref, d_ref, w_ref, ys_in,
               ys_ref):
    _mlp_body(be_ref, act_ref, xs_ref, u_ref, g_ref, d_ref, w_ref, ys_ref)


def _grouped_mlp_part(off, be, act, xs, u, g, d3, ws, ys_in=None):
    gb = off * HNB
    in_specs = [
        pl.BlockSpec((BLK, HID), lambda b, be, act: (b, 0)),
        pl.BlockSpec((INTER, HID), lambda b, be, act: (be[b + gb], 0)),
        pl.BlockSpec((INTER, HID), lambda b, be, act: (be[b + gb], 0)),
        pl.BlockSpec((1, INTER, HID), lambda b, be, act: (be[b + gb], 0, 0)),
        pl.BlockSpec((BLK, 1), lambda b, be, act: (b + gb, 0)),
    ]
    args = [be, act, xs, u, g, d3, ws]
    body = _mlp_body
    aliases = {}
    if ys_in is not None:
        in_specs.append(pl.BlockSpec(memory_space=pl.ANY))
        args.append(ys_in)
        body = _mlp_body2
        aliases = {7: 0}
    grid_spec = pltpu.PrefetchScalarGridSpec(
        num_scalar_prefetch=2,
        grid=(HNB,),
        in_specs=in_specs,
        out_specs=pl.BlockSpec((BLK, HID), lambda b, be, act: (b + gb, 0)),
    )
    return pl.pallas_call(
        body,
        grid_spec=grid_spec,
        out_shape=jax.ShapeDtypeStruct((NSLOT, HID), _F32),
        input_output_aliases=aliases,
        compiler_params=pltpu.CompilerParams(
            dimension_semantics=("arbitrary",)),
    )(*args)


@functools.cache
def _build_sc_combine():
  @functools.partial(
      pl.kernel,
      out_type=jax.ShapeDtypeStruct((BS, HID), _F32),
      mesh=_sc_mesh(),
      scratch_types=[
          pltpu.VMEM((TOPK * TOK_PER_W,), _I32),
          pltpu.VMEM((16, HID), _F32),
          pltpu.VMEM((16, HID), _F32),
          pltpu.VMEM((8, HID), _F32),
          pltpu.SemaphoreType.DMA,
          pltpu.SemaphoreType.DMA,
      ],
      compiler_params=pltpu.CompilerParams(needs_layout_passes=False),
  )
  def _sc_combine(slots_hbm, ys_hbm, out_hbm, sl_v, bufA, bufB, ob, semA,
                  semB):
    wid = lax.axis_index("s") * 2 + lax.axis_index("c")
    t0 = wid * TOK_PER_W
    pltpu.sync_copy(slots_hbm.at[pl.ds(TOPK * t0, TOPK * TOK_PER_W)], sl_v)
    bufs = (bufA, bufB)
    sems = (semA, semB)
    nch = TOK_PER_W // 8
    h = pltpu.async_copy(ys_hbm.at[sl_v.at[pl.ds(0, 16)]], bufs[0], sems[0])
    for c in range(nch):
        h.wait()
        if c + 1 < nch:
            h = pltpu.async_copy(
                ys_hbm.at[sl_v.at[pl.ds((c + 1) * 16, 16)]],
                bufs[(c + 1) % 2], sems[(c + 1) % 2])
        buf = bufs[c % 2]
        for r in range(8):
            def vbody(v, carry, _r=r, _buf=buf):
                a = _buf[2 * _r, pl.ds(v * 16, 16)]
                bq = _buf[2 * _r + 1, pl.ds(v * 16, 16)]
                ob[_r, pl.ds(v * 16, 16)] = a + bq
                return carry

            lax.fori_loop(0, HID // 16, vbody, 0)
        pltpu.sync_copy(ob, out_hbm.at[pl.ds(t0 + c * 8, 8)])

  return _sc_combine


def kernel(hid, gate_w, u, g, d):
    x = hid.reshape(BS, HID)
    slots2, wts2, be2, act2 = _router_dispatch(x, gate_w)
    slots_flat = slots2.reshape(PAIRS)
    w_flat = wts2.reshape(PAIRS)
    be = be2.reshape(NB)
    act = act2.reshape(NB)
    tok, ws = _build_sc_scatter()(slots_flat, w_flat)
    d3 = jnp.transpose(d.reshape(HID, NE, INTER), (1, 2, 0))
    ws2 = ws.reshape(NSLOT, 1)
    xs0 = _build_sc_gather(0)(tok, x)
    xs1 = _build_sc_gather(1)(tok, x)
    ys = _grouped_mlp_part(0, be, act, xs0, u, g, d3, ws2)
    ys = _grouped_mlp_part(1, be, act, xs1, u, g, d3, ws2, ys_in=ys)
    out = _build_sc_combine()(slots_flat, ys)
    return out.reshape(1, BS, HID)


# speedup vs baseline: 1.2969x; 1.0470x over previous
import functools

import jax
import jax.numpy as jnp
from jax import lax
from jax.experimental import pallas as pl
from jax.experimental.pallas import tpu as pltpu
from jax.experimental.pallas import tpu_sc as plsc

HID = 2048
BS = 2048
NE = 16
TOPK = 2
INTER = 448
PAIRS = BS * TOPK
BLK = 64
NB = PAIRS // BLK + NE
NSLOT = NB * BLK
NWORK = 32
SLOT_PER_W = NSLOT // NWORK
GCH = 16
TOK_PER_W = BS // NWORK
NSPLIT = 2
HNB = NB // NSPLIT
HSLOT = NSLOT // NSPLIT
HALF_PER_W = HSLOT // NWORK

@functools.cache
def _sc_mesh():
    return plsc.VectorSubcoreMesh(
        core_axis_name="c", subcore_axis_name="s", num_cores=2,
        num_subcores=16)


_F32 = jnp.float32
_I32 = jnp.int32


def _router_body(x_ref, gw_ref, slots_ref, wts_ref, be_ref, act_ref):
    x = x_ref[...]
    gw = gw_ref[...]
    logits = lax.dot_general(x, gw, (((1,), (1,)), ((), ())),
                             preferred_element_type=_F32)
    m = jnp.max(logits, axis=1, keepdims=True)
    ex = jnp.exp(logits - m)
    rw = ex / jnp.sum(ex, axis=1, keepdims=True)

    ids = lax.broadcasted_iota(_I32, (BS, NE), 1)
    m1 = jnp.max(rw, axis=1, keepdims=True)
    e1 = jnp.min(jnp.where(rw == m1, ids, NE), axis=1, keepdims=True)
    oh1 = (ids == e1)
    v1 = jnp.sum(jnp.where(oh1, rw, 0.0), axis=1, keepdims=True)
    rw2 = jnp.where(oh1, -1.0, rw)
    m2 = jnp.max(rw2, axis=1, keepdims=True)
    e2 = jnp.min(jnp.where(rw2 == m2, ids, NE), axis=1, keepdims=True)
    oh2 = (ids == e2)
    v2 = jnp.sum(jnp.where(oh2, rw, 0.0), axis=1, keepdims=True)
    s = v1 + v2
    w1 = v1 / s
    w2 = v2 / s

    ind = oh1.astype(_F32) + oh2.astype(_F32)
    ri = lax.broadcasted_iota(_I32, (BS, BS), 0)
    ci = lax.broadcasted_iota(_I32, (BS, BS), 1)
    ltri = (ri > ci).astype(_F32)
    rank = lax.dot_general(ltri, ind, (((1,), (0,)), ((), ())),
                           preferred_element_type=_F32)
    counts = jnp.sum(ind, axis=0, keepdims=True)
    pc = jnp.floor((counts + (BLK - 1)) / BLK) * BLK
    e16r = lax.broadcasted_iota(_I32, (NE, NE), 0)
    e16c = lax.broadcasted_iota(_I32, (NE, NE), 1)
    eye16 = (e16r == e16c).astype(_F32)
    sut16 = (e16r < e16c).astype(_F32)
    starts = lax.dot_general(pc, sut16, (((1,), (0,)), ((), ())),
                             preferred_element_type=_F32)
    total = jnp.sum(pc, axis=1, keepdims=True)

    rank1 = jnp.sum(jnp.where(oh1, rank, 0.0), axis=1, keepdims=True)
    rank2 = jnp.sum(jnp.where(oh2, rank, 0.0), axis=1, keepdims=True)
    st1 = jnp.sum(jnp.where(oh1, starts, 0.0), axis=1, keepdims=True)
    st2 = jnp.sum(jnp.where(oh2, starts, 0.0), axis=1, keepdims=True)
    slots_ref[:, 0:1] = (st1 + rank1).astype(_I32)
    slots_ref[:, 1:2] = (st2 + rank2).astype(_I32)
    wts_ref[:, 0:1] = w1
    wts_ref[:, 1:2] = w2

    starts_c = lax.dot_general(eye16, starts, (((1,), (1,)), ((), ())),
                               preferred_element_type=_F32)
    pc_c = lax.dot_general(eye16, pc, (((1,), (1,)), ((), ())),
                           preferred_element_type=_F32)
    bpos = (lax.broadcasted_iota(_I32, (NE, NB), 1) * BLK).astype(_F32)
    inb = jnp.logical_and(bpos >= starts_c, bpos < starts_c + pc_c)
    eids = lax.broadcasted_iota(_I32, (NE, NB), 0).astype(_F32)
    bef = jnp.sum(jnp.where(inb, eids, 0.0), axis=0, keepdims=True)
    bpos_r = (lax.broadcasted_iota(_I32, (1, NB), 1) * BLK).astype(_F32)
    actf = (bpos_r < total).astype(_F32)
    be_ref[...] = (bef * actf + (1.0 - actf) * (NE - 1)).astype(_I32)
    act_ref[...] = actf.astype(_I32)


def _router_dispatch(x, gate_w):
    return pl.pallas_call(
        _router_body,
        out_shape=[
            jax.ShapeDtypeStruct((BS, TOPK), _I32),
            jax.ShapeDtypeStruct((BS, TOPK), _F32),
            jax.ShapeDtypeStruct((1, NB), _I32),
            jax.ShapeDtypeStruct((1, NB), _I32),
        ],
    )(x, gate_w)


@functools.cache
def _build_sc_scatter():
  @functools.partial(
      pl.kernel,
      out_type=[
          jax.ShapeDtypeStruct((NSLOT,), _I32),
          jax.ShapeDtypeStruct((NSLOT,), _F32),
      ],
      mesh=_sc_mesh(),
      scratch_types=[
          pltpu.VMEM((PAIRS,), _I32),
          pltpu.VMEM((PAIRS,), _F32),
          pltpu.VMEM((NSLOT,), _I32),
          pltpu.VMEM((NSLOT,), _F32),
      ],
      compiler_params=pltpu.CompilerParams(needs_layout_passes=False),
  )
  def _sc_scatter(slots_hbm, w_hbm, tok_hbm, ws_hbm, slots_v, w_v, tok_v,
                  ws_v):
    wid = lax.axis_index("s") * 2 + lax.axis_index("c")

    @pl.when(wid == 0)
    def _():
        pltpu.sync_copy(slots_hbm, slots_v)
        pltpu.sync_copy(w_hbm, w_v)
        zi = jnp.zeros((16,), _I32)
        zf = jnp.zeros((16,), _F32)

        def zbody(i, carry):
            tok_v[pl.ds(i * 16, 16)] = zi
            ws_v[pl.ds(i * 16, 16)] = zf
            return carry

        lax.fori_loop(0, NSLOT // 16, zbody, 0)

        def sbody(i, carry):
            sl = slots_v[pl.ds(i * 16, 16)]
            wv = w_v[pl.ds(i * 16, 16)]
            j = lax.iota(_I32, 16) + i * 16
            tok = lax.shift_right_logical(j, 1)
            plsc.store_scatter(tok_v, [sl], tok)
            plsc.store_scatter(ws_v, [sl], wv)
            return carry

        lax.fori_loop(0, PAIRS // 16, sbody, 0)
        pltpu.sync_copy(tok_v, tok_hbm)
        pltpu.sync_copy(ws_v, ws_hbm)

  return _sc_scatter


@functools.cache
def _build_sc_gather(off):
  @functools.partial(
      pl.kernel,
      out_type=jax.ShapeDtypeStruct((HSLOT, HID), _F32),
      mesh=_sc_mesh(),
      scratch_types=[
          pltpu.VMEM((HALF_PER_W,), _I32),
          pltpu.VMEM((GCH, HID), _F32),
          pltpu.VMEM((GCH, HID), _F32),
          pltpu.SemaphoreType.DMA,
          pltpu.SemaphoreType.DMA,
      ],
      compiler_params=pltpu.CompilerParams(needs_layout_passes=False),
  )
  def _sc_gather(tok_hbm, x_hbm, xs_hbm, tok_v, buf0, buf1, sem0, sem1):
    wid = lax.axis_index("s") * 2 + lax.axis_index("c")
    base = wid * HALF_PER_W
    pltpu.sync_copy(tok_hbm.at[pl.ds(off * HSLOT + base, HALF_PER_W)], tok_v)
    bufs = (buf0, buf1)
    sems = (sem0, sem1)
    nch = HALF_PER_W // GCH
    h = pltpu.async_copy(x_hbm.at[tok_v.at[pl.ds(0, GCH)]], bufs[0], sems[0])
    for c in range(nch):
        h.wait()
        if c + 1 < nch:
            h = pltpu.async_copy(
                x_hbm.at[tok_v.at[pl.ds((c + 1) * GCH, GCH)]],
                bufs[(c + 1) % 2], sems[(c + 1) % 2])
        pltpu.sync_copy(bufs[c % 2], xs_hbm.at[pl.ds(base + c * GCH, GCH)])

  return _sc_gather


def _mlp_body(be_ref, act_ref, xs_ref, u_ref, g_ref, d_ref, w_ref, ys_ref):
    b = pl.program_id(0)

    @pl.when(act_ref[b] == 1)
    def _():
        xb = xs_ref[...]
        hg = lax.dot_general(xb, g_ref[...], (((1,), (1,)), ((), ())),
                             preferred_element_type=_F32,
                             precision=lax.Precision.DEFAULT)
        hu = lax.dot_general(xb, u_ref[...], (((1,), (1,)), ((), ())),
                             preferred_element_type=_F32,
                             precision=lax.Precision.DEFAULT)
        hact = (hg * (1.0 / (1.0 + jnp.exp(-hg)))) * hu
        y = lax.dot_general(hact, d_ref[0], (((1,), (0,)), ((), ())),
                            preferred_element_type=_F32,
                            precision=lax.Precision.DEFAULT)
        ys_ref[...] = y * w_ref[...]


def _mlp_body2(be_ref, act_ref, xs_ref, u_ref, g_ref, d_ref, w_ref, ys_in,
               ys_ref):
    _mlp_body(be_ref, act_ref, xs_ref, u_ref, g_ref, d_ref, w_ref, ys_ref)


def _grouped_mlp_part(off, be, act, xs, u, g, d3, ws, ys_in=None):
    gb = off * HNB
    in_specs = [
        pl.BlockSpec((BLK, HID), lambda b, be, act: (b, 0)),
        pl.BlockSpec((INTER, HID), lambda b, be, act: (be[b + gb], 0)),
        pl.BlockSpec((INTER, HID), lambda b, be, act: (be[b + gb], 0)),
        pl.BlockSpec((1, INTER, HID), lambda b, be, act: (be[b + gb], 0, 0)),
        pl.BlockSpec((BLK, 1), lambda b, be, act: (b + gb, 0)),
    ]
    args = [be, act, xs, u, g, d3, ws]
    body = _mlp_body
    aliases = {}
    if ys_in is not None:
        in_specs.append(pl.BlockSpec(memory_space=pl.ANY))
        args.append(ys_in)
        body = _mlp_body2
        aliases = {7: 0}
    grid_spec = pltpu.PrefetchScalarGridSpec(
        num_scalar_prefetch=2,
        grid=(HNB,),
        in_specs=in_specs,
        out_specs=pl.BlockSpec((BLK, HID), lambda b, be, act: (b + gb, 0)),
    )
    return pl.pallas_call(
        body,
        grid_spec=grid_spec,
        out_shape=jax.ShapeDtypeStruct((NSLOT, HID), _F32),
        input_output_aliases=aliases,
        compiler_params=pltpu.CompilerParams(
            dimension_semantics=("arbitrary",)),
    )(*args)


@functools.cache
def _build_sc_combine():
  @functools.partial(
      pl.kernel,
      out_type=jax.ShapeDtypeStruct((BS, HID), _F32),
      mesh=_sc_mesh(),
      scratch_types=[
          pltpu.VMEM((TOPK * TOK_PER_W,), _I32),
          pltpu.VMEM((16, HID), _F32),
          pltpu.VMEM((16, HID), _F32),
          pltpu.VMEM((8, HID), _F32),
          pltpu.SemaphoreType.DMA,
          pltpu.SemaphoreType.DMA,
      ],
      compiler_params=pltpu.CompilerParams(needs_layout_passes=False),
  )
  def _sc_combine(slots_hbm, ys_hbm, out_hbm, sl_v, bufA, bufB, ob, semA,
                  semB):
    wid = lax.axis_index("s") * 2 + lax.axis_index("c")
    t0 = wid * TOK_PER_W
    pltpu.sync_copy(slots_hbm.at[pl.ds(TOPK * t0, TOPK * TOK_PER_W)], sl_v)
    bufs = (bufA, bufB)
    sems = (semA, semB)
    nch = TOK_PER_W // 8
    h = pltpu.async_copy(ys_hbm.at[sl_v.at[pl.ds(0, 16)]], bufs[0], sems[0])
    for c in range(nch):
        h.wait()
        if c + 1 < nch:
            h = pltpu.async_copy(
                ys_hbm.at[sl_v.at[pl.ds((c + 1) * 16, 16)]],
                bufs[(c + 1) % 2], sems[(c + 1) % 2])
        buf = bufs[c % 2]
        for r in range(8):
            def vbody(v, carry, _r=r, _buf=buf):
                a = _buf[2 * _r, pl.ds(v * 16, 16)]
                bq = _buf[2 * _r + 1, pl.ds(v * 16, 16)]
                ob[_r, pl.ds(v * 16, 16)] = a + bq
                return carry

            lax.fori_loop(0, HID // 16, vbody, 0)
        pltpu.sync_copy(ob, out_hbm.at[pl.ds(t0 + c * 8, 8)])

  return _sc_combine


def kernel(hid, gate_w, u, g, d):
    x = hid.reshape(BS, HID)
    slots2, wts2, be2, act2 = _router_dispatch(x, gate_w)
    slots_flat = slots2.reshape(PAIRS)
    w_flat = wts2.reshape(PAIRS)
    be = be2.reshape(NB)
    act = act2.reshape(NB)
    tok, ws = _build_sc_scatter()(slots_flat, w_flat)
    d3 = jnp.transpose(d.reshape(HID, NE, INTER), (1, 2, 0))
    ws2 = ws.reshape(NSLOT, 1)
    xs0 = _build_sc_gather(0)(tok, x)
    xs1 = _build_sc_gather(1)(tok, x)
    ys = _grouped_mlp_part(0, be, act, xs0, u, g, d3, ws2)
    ys = _grouped_mlp_part(1, be, act, xs1, u, g, d3, ws2, ys_in=ys)
    out = _build_sc_combine()(slots_flat, ys)
    return out.reshape(1, BS, HID)
